# baseline (speedup 1.0000x reference)
import jax
import jax.numpy as jnp
from jax.experimental import pallas as pl
from jax.experimental.pallas import tpu as pltpu

_R, _C = 4096, 4096
_BR = 256
_G = _R // _BR


def _mse_body(pred_ref, ref_ref, mask_ref, out_ref, acc_ref):
    i = pl.program_id(0)

    @pl.when(i == 0)
    def _init():
        acc_ref[0] = 0.0
        acc_ref[1] = 0.0

    keep = jnp.logical_not(mask_ref[...])
    d = jnp.where(keep, pred_ref[...] - ref_ref[...], 0.0)
    acc_ref[0] += jnp.sum(d * d)
    acc_ref[1] += jnp.sum(keep.astype(jnp.float32))

    @pl.when(i == _G - 1)
    def _fin():
        out_ref[...] = jnp.full((1, 1), acc_ref[0] / acc_ref[1], jnp.float32)


def kernel(pred, ref, mask):
    p = pred.reshape(_R, _C)
    r = ref.reshape(_R, _C)
    m = mask.reshape(_R, _C)
    out = pl.pallas_call(
        _mse_body,
        grid=(_G,),
        in_specs=[
            pl.BlockSpec((_BR, _C), lambda i: (i, 0)),
            pl.BlockSpec((_BR, _C), lambda i: (i, 0)),
            pl.BlockSpec((_BR, _C), lambda i: (i, 0)),
        ],
        out_specs=pl.BlockSpec((1, 1), lambda i: (0, 0)),
        out_shape=jax.ShapeDtypeStruct((1, 1), jnp.float32),
        scratch_shapes=[pltpu.SMEM((2,), jnp.float32)],
    )(p, r, m)
    return out[0, 0]


# speedup vs baseline: 1.0199x; 1.0199x over previous
import jax
import jax.numpy as jnp
from jax.experimental import pallas as pl
from jax.experimental.pallas import tpu as pltpu

_R, _C = 4096, 4096
_BR = 128
_G = _R // _BR


def _mse_body(pred_ref, ref_ref, mask_ref, out_ref, acc_ref):
    i = pl.program_id(0)

    @pl.when(i == 0)
    def _init():
        acc_ref[0] = 0.0
        acc_ref[1] = 0.0

    keep = jnp.logical_not(mask_ref[...])
    d = jnp.where(keep, pred_ref[...] - ref_ref[...], 0.0)
    acc_ref[0] += jnp.sum(d * d)
    acc_ref[1] += jnp.sum(keep.astype(jnp.float32))

    @pl.when(i == _G - 1)
    def _fin():
        out_ref[...] = jnp.full((1, 1), acc_ref[0] / acc_ref[1], jnp.float32)


def kernel(pred, ref, mask):
    p = pred.reshape(_R, _C)
    r = ref.reshape(_R, _C)
    m = mask.reshape(_R, _C)
    out = pl.pallas_call(
        _mse_body,
        grid=(_G,),
        in_specs=[
            pl.BlockSpec((_BR, _C), lambda i: (i, 0)),
            pl.BlockSpec((_BR, _C), lambda i: (i, 0)),
            pl.BlockSpec((_BR, _C), lambda i: (i, 0)),
        ],
        out_specs=pl.BlockSpec((1, 1), lambda i: (0, 0)),
        out_shape=jax.ShapeDtypeStruct((1, 1), jnp.float32),
        scratch_shapes=[pltpu.SMEM((2,), jnp.float32)],
    )(p, r, m)
    return out[0, 0]


# speedup vs baseline: 1.3376x; 1.3114x over previous
import jax
import jax.numpy as jnp
from jax.experimental import pallas as pl
from jax.experimental.pallas import tpu as pltpu

_R, _C = 4096, 4096
_BR = 128
_G = _R // _BR


def _mse_body(pred_ref, ref_ref, mask_ref, out_ref, acc_ref):
    i = pl.program_id(0)

    @pl.when(i == 0)
    def _init():
        acc_ref[0] = 0.0
        acc_ref[1] = 0.0

    keep = mask_ref[...] == 0
    d = jnp.where(keep, pred_ref[...] - ref_ref[...], 0.0)
    acc_ref[0] += jnp.sum(d * d)
    acc_ref[1] += jnp.sum(keep.astype(jnp.float32))

    @pl.when(i == _G - 1)
    def _fin():
        out_ref[...] = jnp.full((1, 1), acc_ref[0] / acc_ref[1], jnp.float32)


def kernel(pred, ref, mask):
    p = pred.reshape(_R, _C)
    r = ref.reshape(_R, _C)
    m = mask.view(jnp.int8).reshape(_R, _C)
    out = pl.pallas_call(
        _mse_body,
        grid=(_G,),
        in_specs=[
            pl.BlockSpec((_BR, _C), lambda i: (i, 0)),
            pl.BlockSpec((_BR, _C), lambda i: (i, 0)),
            pl.BlockSpec((_BR, _C), lambda i: (i, 0)),
        ],
        out_specs=pl.BlockSpec((1, 1), lambda i: (0, 0)),
        out_shape=jax.ShapeDtypeStruct((1, 1), jnp.float32),
        scratch_shapes=[pltpu.SMEM((2,), jnp.float32)],
    )(p, r, m)
    return out[0, 0]


# speedup vs baseline: 1.4549x; 1.0877x over previous
import jax
import jax.numpy as jnp
from jax.experimental import pallas as pl
from jax.experimental.pallas import tpu as pltpu

_R, _C = 4096, 4096
_BR = 256
_G = _R // _BR


def _mse_body(pred_ref, ref_ref, mask_ref, out_ref, acc_ref):
    i = pl.program_id(0)

    @pl.when(i == 0)
    def _init():
        acc_ref[0] = 0.0
        acc_ref[1] = 0.0

    keep = mask_ref[...] == 0
    d = jnp.where(keep, pred_ref[...] - ref_ref[...], 0.0)
    acc_ref[0] += jnp.sum(d * d)
    acc_ref[1] += jnp.sum(keep.astype(jnp.float32))

    @pl.when(i == _G - 1)
    def _fin():
        out_ref[...] = jnp.full((1, 1), acc_ref[0] / acc_ref[1], jnp.float32)


def kernel(pred, ref, mask):
    p = pred.reshape(_R, _C)
    r = ref.reshape(_R, _C)
    m = mask.view(jnp.int8).reshape(_R, _C)
    out = pl.pallas_call(
        _mse_body,
        grid=(_G,),
        in_specs=[
            pl.BlockSpec((_BR, _C), lambda i: (i, 0)),
            pl.BlockSpec((_BR, _C), lambda i: (i, 0)),
            pl.BlockSpec((_BR, _C), lambda i: (i, 0)),
        ],
        out_specs=pl.BlockSpec((1, 1), lambda i: (0, 0)),
        out_shape=jax.ShapeDtypeStruct((1, 1), jnp.float32),
        scratch_shapes=[pltpu.SMEM((2,), jnp.float32)],
    )(p, r, m)
    return out[0, 0]


# speedup vs baseline: 1.4905x; 1.0244x over previous
import jax
import jax.numpy as jnp
from jax.experimental import pallas as pl
from jax.experimental.pallas import tpu as pltpu

_R, _C = 4096, 4096
_BR = 512
_G = _R // _BR


def _mse_body(pred_ref, ref_ref, mask_ref, out_ref, acc_ref):
    i = pl.program_id(0)

    @pl.when(i == 0)
    def _init():
        acc_ref[0] = 0.0
        acc_ref[1] = 0.0

    keep = mask_ref[...] == 0
    d = jnp.where(keep, pred_ref[...] - ref_ref[...], 0.0)
    acc_ref[0] += jnp.sum(d * d)
    acc_ref[1] += jnp.sum(keep.astype(jnp.float32))

    @pl.when(i == _G - 1)
    def _fin():
        out_ref[...] = jnp.full((1, 1), acc_ref[0] / acc_ref[1], jnp.float32)


def kernel(pred, ref, mask):
    p = pred.reshape(_R, _C)
    r = ref.reshape(_R, _C)
    m = mask.view(jnp.int8).reshape(_R, _C)
    out = pl.pallas_call(
        _mse_body,
        grid=(_G,),
        in_specs=[
            pl.BlockSpec((_BR, _C), lambda i: (i, 0)),
            pl.BlockSpec((_BR, _C), lambda i: (i, 0)),
            pl.BlockSpec((_BR, _C), lambda i: (i, 0)),
        ],
        out_specs=pl.BlockSpec((1, 1), lambda i: (0, 0)),
        out_shape=jax.ShapeDtypeStruct((1, 1), jnp.float32),
        scratch_shapes=[pltpu.SMEM((2,), jnp.float32)],
    )(p, r, m)
    return out[0, 0]
